# baseline (speedup 1.0000x reference)
import functools

import jax
import jax.numpy as jnp
from jax.experimental import pallas as pl


def _bn_ac(s_ref, ss_ref, g_ref, be_ref, cnt):
    mean = jnp.sum(s_ref[...], axis=1, keepdims=True) * (1.0 / cnt)
    var = jnp.maximum(
        jnp.sum(ss_ref[...], axis=1, keepdims=True) * (1.0 / cnt) - mean * mean,
        0.0,
    )
    a = g_ref[...] * jax.lax.rsqrt(var + 1e-5)
    c = be_ref[...] - mean * a
    return a, c


def _fused(
    x2t_ref, x1_ref, p2_ref, p1_ref, w1_ref, h1in_ref, g1_ref, be1_ref,
    w2_ref, h2in_ref, g2_ref, be2_ref,
    o_ref, h1_ref, s1_ref, ss1_ref, h2_ref, s2_ref, ss2_ref, *, cnt,
):
    p = pl.program_id(0)
    b = pl.program_id(1)

    @pl.when(jnp.logical_and(p == 0, b == 0))
    def _init():
        s1_ref[...] = jnp.zeros_like(s1_ref)
        ss1_ref[...] = jnp.zeros_like(ss1_ref)
        s2_ref[...] = jnp.zeros_like(s2_ref)
        ss2_ref[...] = jnp.zeros_like(ss2_ref)

    @pl.when(p == 0)
    def _phase0():
        M = x2t_ref.shape[0]
        C2 = p2_ref.shape[0]
        x2 = x2t_ref[...]
        x1 = x1_ref[...]
        n2 = jnp.sum(x2 * x2, axis=1, keepdims=True)
        n1 = jnp.sum(x1 * x1, axis=0, keepdims=True)
        x2a = jnp.concatenate([x2, n2, jnp.ones_like(n2)], axis=1)
        x1a = jnp.concatenate([-2.0 * x1, jnp.ones_like(n1), n1], axis=0)
        d = jax.lax.dot_general(
            x2a,
            x1a,
            (((1,), (0,)), ((), ())),
            precision=jax.lax.Precision.HIGHEST,
            preferred_element_type=jnp.float32,
        )

        inf = jnp.float32(jnp.inf)
        m1 = jnp.min(d, axis=0, keepdims=True)
        m2 = jnp.min(jnp.where(d > m1, d, inf), axis=0, keepdims=True)
        m3 = jnp.min(jnp.where(d > m2, d, inf), axis=0, keepdims=True)

        r1 = 1.0 / jnp.maximum(m1, 1e-10)
        r2 = 1.0 / jnp.maximum(m2, 1e-10)
        r3 = 1.0 / jnp.maximum(m3, 1e-10)
        rs = r1 + r2 + r3
        st = jnp.where(d <= m3, 1.0 / (jnp.maximum(d, 1e-10) * rs), 0.0)

        interp = jnp.dot(p2_ref[...], st, preferred_element_type=jnp.float32)
        h = jnp.dot(w1_ref[:, :C2], interp, preferred_element_type=jnp.float32)
        h = h + jnp.dot(
            w1_ref[:, C2:], p1_ref[...], preferred_element_type=jnp.float32
        )
        h1_ref[...] = h.astype(h1_ref.dtype)

        hh = h * h
        nb = h.shape[1]
        s1_ref[...] += sum(h[:, j * 128 : (j + 1) * 128] for j in range(nb // 128))
        ss1_ref[...] += sum(hh[:, j * 128 : (j + 1) * 128] for j in range(nb // 128))

    @pl.when(p == 1)
    def _phase1():
        a, c = _bn_ac(s1_ref, ss1_ref, g1_ref, be1_ref, cnt)
        hn = jnp.maximum(a * h1in_ref[...].astype(jnp.float32) + c, 0.0)
        h2 = jnp.dot(w2_ref[...], hn, preferred_element_type=jnp.float32)
        h2_ref[...] = h2.astype(h2_ref.dtype)

        hh = h2 * h2
        nb = h2.shape[1]
        s2_ref[...] += sum(h2[:, j * 128 : (j + 1) * 128] for j in range(nb // 128))
        ss2_ref[...] += sum(hh[:, j * 128 : (j + 1) * 128] for j in range(nb // 128))

    @pl.when(p == 2)
    def _phase2():
        a, c = _bn_ac(s2_ref, ss2_ref, g2_ref, be2_ref, cnt)
        o_ref[...] = jnp.maximum(a * h2in_ref[...].astype(jnp.float32) + c, 0.0)


def kernel(xyz1, xyz2, points1, points2, W1, b1, g1, be1, W2, b2, g2, be2):
    B, _, N = xyz1.shape
    M = xyz2.shape[2]
    C1 = points1.shape[1]
    C2 = points2.shape[1]
    H1 = W1.shape[0]
    H2 = W2.shape[0]
    cnt = B * N
    last = B - 1

    x2t = jnp.transpose(xyz2, (0, 2, 1))
    h1_buf = jnp.zeros((B, H1, N), jnp.bfloat16)
    h2_buf = jnp.zeros((B, H2, N), jnp.bfloat16)

    def ph0(p, b):
        return jnp.where(p == 0, b, last)

    out = pl.pallas_call(
        functools.partial(_fused, cnt=cnt),
        grid=(3, B),
        in_specs=[
            pl.BlockSpec((None, M, 3), lambda p, b: (ph0(p, b), 0, 0)),
            pl.BlockSpec((None, 3, N), lambda p, b: (ph0(p, b), 0, 0)),
            pl.BlockSpec((None, C2, M), lambda p, b: (ph0(p, b), 0, 0)),
            pl.BlockSpec((None, C1, N), lambda p, b: (ph0(p, b), 0, 0)),
            pl.BlockSpec((H1, C2 + C1), lambda p, b: (0, 0)),
            pl.BlockSpec((None, H1, N), lambda p, b: (jnp.where(p == 1, b, last), 0, 0)),
            pl.BlockSpec((H1, 1), lambda p, b: (0, 0)),
            pl.BlockSpec((H1, 1), lambda p, b: (0, 0)),
            pl.BlockSpec((H2, H1), lambda p, b: (0, 0)),
            pl.BlockSpec((None, H2, N), lambda p, b: (jnp.where(p == 2, b, last), 0, 0)),
            pl.BlockSpec((H2, 1), lambda p, b: (0, 0)),
            pl.BlockSpec((H2, 1), lambda p, b: (0, 0)),
        ],
        out_specs=[
            pl.BlockSpec((None, H2, N), lambda p, b: (jnp.where(p == 2, b, 0), 0, 0)),
            pl.BlockSpec((None, H1, N), lambda p, b: (jnp.where(p == 0, b, 0), 0, 0)),
            pl.BlockSpec((H1, 128), lambda p, b: (0, 0)),
            pl.BlockSpec((H1, 128), lambda p, b: (0, 0)),
            pl.BlockSpec((None, H2, N), lambda p, b: (jnp.where(p == 1, b, 0), 0, 0)),
            pl.BlockSpec((H2, 128), lambda p, b: (0, 0)),
            pl.BlockSpec((H2, 128), lambda p, b: (0, 0)),
        ],
        out_shape=[
            jax.ShapeDtypeStruct((B, H2, N), jnp.float32),
            jax.ShapeDtypeStruct((B, H1, N), jnp.bfloat16),
            jax.ShapeDtypeStruct((H1, 128), jnp.float32),
            jax.ShapeDtypeStruct((H1, 128), jnp.float32),
            jax.ShapeDtypeStruct((B, H2, N), jnp.bfloat16),
            jax.ShapeDtypeStruct((H2, 128), jnp.float32),
            jax.ShapeDtypeStruct((H2, 128), jnp.float32),
        ],
        input_output_aliases={5: 1, 9: 4},
    )(
        x2t, xyz1, points2, points1, W1, h1_buf, g1[:, None], be1[:, None],
        W2, h2_buf, g2[:, None], be2[:, None],
    )[0]

    return out


# speedup vs baseline: 1.0562x; 1.0562x over previous
import functools

import jax
import jax.numpy as jnp
from jax.experimental import pallas as pl

_NB1 = 4096
_NB2 = 4096
_NB3 = 4096


def _k1(x2t_ref, x1_ref, p2_ref, p1_ref, w1_ref, h1_ref, s_ref, ss_ref):
    b = pl.program_id(0)
    nt = pl.program_id(1)
    M = x2t_ref.shape[0]
    C2 = p2_ref.shape[0]

    x2 = x2t_ref[...]
    x1 = x1_ref[...]
    n2 = jnp.sum(x2 * x2, axis=1, keepdims=True)
    n1 = jnp.sum(x1 * x1, axis=0, keepdims=True)
    x2a = jnp.concatenate([x2, n2, jnp.ones_like(n2)], axis=1)
    x1a = jnp.concatenate([-2.0 * x1, jnp.ones_like(n1), n1], axis=0)
    d = jax.lax.dot_general(
        x2a,
        x1a,
        (((1,), (0,)), ((), ())),
        precision=jax.lax.Precision.HIGHEST,
        preferred_element_type=jnp.float32,
    )

    inf = jnp.float32(jnp.inf)
    m1 = jnp.min(d, axis=0, keepdims=True)
    m2 = jnp.min(jnp.where(d > m1, d, inf), axis=0, keepdims=True)
    m3 = jnp.min(jnp.where(d > m2, d, inf), axis=0, keepdims=True)

    r1 = 1.0 / jnp.maximum(m1, 1e-10)
    r2 = 1.0 / jnp.maximum(m2, 1e-10)
    r3 = 1.0 / jnp.maximum(m3, 1e-10)
    rs = r1 + r2 + r3
    st = jnp.where(d <= m3, 1.0 / (jnp.maximum(d, 1e-10) * rs), 0.0)

    interp = jnp.dot(p2_ref[...], st, preferred_element_type=jnp.float32)
    h = jnp.dot(w1_ref[:, :C2], interp, preferred_element_type=jnp.float32)
    h = h + jnp.dot(w1_ref[:, C2:], p1_ref[...], preferred_element_type=jnp.float32)
    h1_ref[...] = h.astype(h1_ref.dtype)

    @pl.when(jnp.logical_and(b == 0, nt == 0))
    def _init():
        s_ref[...] = jnp.zeros_like(s_ref)
        ss_ref[...] = jnp.zeros_like(ss_ref)

    hh = h * h
    nb = h.shape[1]
    s_ref[...] += sum(h[:, j * 128 : (j + 1) * 128] for j in range(nb // 128))
    ss_ref[...] += sum(hh[:, j * 128 : (j + 1) * 128] for j in range(nb // 128))


def _bn_ac(s_ref, ss_ref, g_ref, be_ref, cnt):
    mean = jnp.sum(s_ref[...], axis=1, keepdims=True) * (1.0 / cnt)
    var = jnp.maximum(
        jnp.sum(ss_ref[...], axis=1, keepdims=True) * (1.0 / cnt) - mean * mean,
        0.0,
    )
    a = g_ref[...] * jax.lax.rsqrt(var + 1e-5)
    c = be_ref[...] - mean * a
    return a, c


def _k2(h1_ref, sin_ref, ssin_ref, g_ref, be_ref, w2_ref, h2_ref, s_ref, ss_ref, *, cnt):
    b = pl.program_id(0)
    nt = pl.program_id(1)
    a, c = _bn_ac(sin_ref, ssin_ref, g_ref, be_ref, cnt)
    hn = jnp.maximum(a * h1_ref[...].astype(jnp.float32) + c, 0.0)
    h2 = jnp.dot(w2_ref[...], hn, preferred_element_type=jnp.float32)
    h2_ref[...] = h2.astype(h2_ref.dtype)

    @pl.when(jnp.logical_and(b == 0, nt == 0))
    def _init():
        s_ref[...] = jnp.zeros_like(s_ref)
        ss_ref[...] = jnp.zeros_like(ss_ref)

    hh = h2 * h2
    nb = h2.shape[1]
    s_ref[...] += sum(h2[:, j * 128 : (j + 1) * 128] for j in range(nb // 128))
    ss_ref[...] += sum(hh[:, j * 128 : (j + 1) * 128] for j in range(nb // 128))


def _k3(h2_ref, sin_ref, ssin_ref, g_ref, be_ref, o_ref, *, cnt):
    a, c = _bn_ac(sin_ref, ssin_ref, g_ref, be_ref, cnt)
    o_ref[...] = jnp.maximum(a * h2_ref[...].astype(jnp.float32) + c, 0.0)


def kernel(xyz1, xyz2, points1, points2, W1, b1, g1, be1, W2, b2, g2, be2):
    B, _, N = xyz1.shape
    M = xyz2.shape[2]
    C1 = points1.shape[1]
    C2 = points2.shape[1]
    H1 = W1.shape[0]
    H2 = W2.shape[0]
    cnt = B * N
    nb1 = min(_NB1, N)
    nb2 = min(_NB2, N)
    nb3 = min(_NB3, N)

    x2t = jnp.transpose(xyz2, (0, 2, 1))

    h1, s1, ss1 = pl.pallas_call(
        _k1,
        grid=(B, N // nb1),
        in_specs=[
            pl.BlockSpec((None, M, 3), lambda b, n: (b, 0, 0)),
            pl.BlockSpec((None, 3, nb1), lambda b, n: (b, 0, n)),
            pl.BlockSpec((None, C2, M), lambda b, n: (b, 0, 0)),
            pl.BlockSpec((None, C1, nb1), lambda b, n: (b, 0, n)),
            pl.BlockSpec((H1, C2 + C1), lambda b, n: (0, 0)),
        ],
        out_specs=[
            pl.BlockSpec((None, H1, nb1), lambda b, n: (b, 0, n)),
            pl.BlockSpec((H1, 128), lambda b, n: (0, 0)),
            pl.BlockSpec((H1, 128), lambda b, n: (0, 0)),
        ],
        out_shape=[
            jax.ShapeDtypeStruct((B, H1, N), jnp.bfloat16),
            jax.ShapeDtypeStruct((H1, 128), jnp.float32),
            jax.ShapeDtypeStruct((H1, 128), jnp.float32),
        ],
    )(x2t, xyz1, points2, points1, W1)

    h2, s2, ss2 = pl.pallas_call(
        functools.partial(_k2, cnt=cnt),
        grid=(B, N // nb2),
        in_specs=[
            pl.BlockSpec((None, H1, nb2), lambda b, n: (b, 0, n)),
            pl.BlockSpec((H1, 128), lambda b, n: (0, 0)),
            pl.BlockSpec((H1, 128), lambda b, n: (0, 0)),
            pl.BlockSpec((H1, 1), lambda b, n: (0, 0)),
            pl.BlockSpec((H1, 1), lambda b, n: (0, 0)),
            pl.BlockSpec((H2, H1), lambda b, n: (0, 0)),
        ],
        out_specs=[
            pl.BlockSpec((None, H2, nb2), lambda b, n: (b, 0, n)),
            pl.BlockSpec((H2, 128), lambda b, n: (0, 0)),
            pl.BlockSpec((H2, 128), lambda b, n: (0, 0)),
        ],
        out_shape=[
            jax.ShapeDtypeStruct((B, H2, N), jnp.bfloat16),
            jax.ShapeDtypeStruct((H2, 128), jnp.float32),
            jax.ShapeDtypeStruct((H2, 128), jnp.float32),
        ],
    )(h1, s1, ss1, g1[:, None], be1[:, None], W2)

    out = pl.pallas_call(
        functools.partial(_k3, cnt=cnt),
        grid=(B, N // nb3),
        in_specs=[
            pl.BlockSpec((None, H2, nb3), lambda b, n: (b, 0, n)),
            pl.BlockSpec((H2, 128), lambda b, n: (0, 0)),
            pl.BlockSpec((H2, 128), lambda b, n: (0, 0)),
            pl.BlockSpec((H2, 1), lambda b, n: (0, 0)),
            pl.BlockSpec((H2, 1), lambda b, n: (0, 0)),
        ],
        out_specs=pl.BlockSpec((None, H2, nb3), lambda b, n: (b, 0, n)),
        out_shape=jax.ShapeDtypeStruct((B, H2, N), jnp.float32),
    )(h2, s2, ss2, g2[:, None], be2[:, None])

    return out
